# baseline (speedup 1.0000x reference)
import functools

import jax
import jax.numpy as jnp
from jax import lax
from jax.experimental import pallas as pl
from jax.experimental.pallas import tpu as pltpu
from jax.experimental.pallas import tpu_sc as plsc

N_USERS = 5000
N_ITEMS = 5000
N_NODES = N_USERS + N_ITEMS
DIM = 256
N_EDGES = 160000
DROP = 0.1
DECAY = 1e-4
BATCH = 4096

NC = 2
NS = 16
L = 16

NW = NC * NS
DW = DIM // NW
BLK = 8000
NBLK = N_EDGES // BLK
SEG_PAD = 5120
VG = DIM // L
ACC = SEG_PAD * DW
TABW = N_USERS * DW

BPB = BATCH // (NC * NS)


def _agg_body(tab_hbm, pack_hbm, vals_hbm, keep_hbm, out_hbm,
              psb, vsb, ksb, dvb, tabv, acc):
    cid = lax.axis_index("c")
    sid = lax.axis_index("s")
    w = cid * NS + sid
    scale = 1.0 / (1.0 - DROP)

    for a in range(2):
        pltpu.sync_copy(tab_hbm.at[pl.ds((a * NW + w) * TABW, TABW)], tabv)

        def z_acc(i, carry):
            acc[pl.ds(i * L, L)] = jnp.zeros((L,), jnp.float32)
            return carry

        lax.fori_loop(0, ACC // L, z_acc, 0)

        def blk_body(b, carry):
            base = b * BLK
            pltpu.sync_copy(pack_hbm.at[pl.ds(a * N_EDGES + base, BLK)], psb)
            pltpu.sync_copy(vals_hbm.at[pl.ds(base, BLK)], vsb)
            pltpu.sync_copy(keep_hbm.at[pl.ds(base, BLK)], ksb)

            def dv_group(g, c2):
                s = pl.ds(g * L, L)
                dvb[s] = vsb[s] * ksb[s] * scale
                return c2

            lax.fori_loop(0, BLK // L, dv_group, 0)

            @plsc.parallel_loop(0, BLK // L, unroll=8)
            def grp_body(g):
                s = pl.ds(g * L, L)
                p16 = psb[s]
                dvv = dvb[s]
                sb8 = jnp.bitwise_and(p16, 0xFFFF)
                db8 = lax.shift_right_logical(p16, 16)
                for d in range(DW):
                    xv = plsc.load_gather(tabv, [sb8 + d])
                    plsc.addupdate_scatter(acc, [db8 + d], xv * dvv)
            return carry

        lax.fori_loop(0, NBLK, blk_body, 0)
        pltpu.sync_copy(acc, out_hbm.at[pl.ds((a * NW + w) * ACC, ACC)])


def _bpr_body(emb_hbm, uidx_hbm, pidx_hbm, nidx_hbm, pd_hbm, nd_hbm, sq_hbm,
              uidx_v, pidx_v, nidx_v, ubuf, pbuf, nbuf, pdv, ndv, sqv, sem):
    cid = lax.axis_index("c")
    sid = lax.axis_index("s")
    wid = sid * NC + cid
    base = wid * BPB

    pltpu.sync_copy(uidx_hbm.at[pl.ds(base, BPB)], uidx_v)
    pltpu.sync_copy(pidx_hbm.at[pl.ds(base, BPB)], pidx_v)
    pltpu.sync_copy(nidx_hbm.at[pl.ds(base, BPB)], nidx_v)
    c1 = pltpu.async_copy(emb_hbm.at[uidx_v], ubuf, sem)
    c2 = pltpu.async_copy(emb_hbm.at[pidx_v], pbuf, sem)
    c3 = pltpu.async_copy(emb_hbm.at[nidx_v], nbuf, sem)
    c1.wait()
    c2.wait()
    c3.wait()

    lane = lax.iota(jnp.int32, L)
    zero = jnp.zeros((L,), jnp.float32)

    def body(i, carry):
        sacc, pvec, nvec = carry
        pacc = zero
        nacc = zero
        for g in range(VG):
            s = pl.ds(g * L, L)
            u = ubuf[i, s]
            p = pbuf[i, s]
            n = nbuf[i, s]
            pacc = pacc + u * p
            nacc = nacc + u * n
            sacc = sacc + u * u + p * p + n * n
        ps = jnp.sum(pacc)
        ns = jnp.sum(nacc)
        j = jnp.bitwise_and(i, L - 1)
        pvec = jnp.where(lane == j, ps, pvec)
        nvec = jnp.where(lane == j, ns, nvec)

        @pl.when(j == L - 1)
        def _flush():
            st = i - (L - 1)
            pdv[pl.ds(st, L)] = pvec
            ndv[pl.ds(st, L)] = nvec

        return (sacc, pvec, nvec)

    sacc, _, _ = lax.fori_loop(0, BPB, body, (zero, zero, zero))
    sqv[...] = sacc
    pltpu.sync_copy(pdv, pd_hbm.at[pl.ds(base, BPB)])
    pltpu.sync_copy(ndv, nd_hbm.at[pl.ds(base, BPB)])
    pltpu.sync_copy(sqv, sq_hbm.at[pl.ds(wid * L, L)])


def _slab_reduce_block(x_ref, o_ref):
    o_ref[...] = jnp.sum(x_ref[0], axis=0)[None]


def _emb_block(x_ref, w_ref, b_ref, o_ref):
    x = x_ref[...]
    y = jnp.dot(x, w_ref[...], preferred_element_type=jnp.float32) + b_ref[...]
    y = jnp.where(y >= 0, y, 0.2 * y)
    n = jnp.sqrt(jnp.sum(y * y, axis=1, keepdims=True))
    o_ref[...] = y / jnp.maximum(n, 1e-12)


def _loss_block(pd_ref, nd_ref, sq_ref, o_ref):
    d = pd_ref[...] - nd_ref[...]
    z = -d
    sp = jnp.maximum(z, 0.0) + jnp.log1p(jnp.exp(-jnp.abs(z)))
    mf = jnp.sum(sp) * (1.0 / BATCH)
    reg = jnp.sum(sq_ref[...]) * 0.5
    o_ref[...] = jnp.broadcast_to(mf + DECAY * reg * (1.0 / BATCH), (1, 1))


def kernel(node_emb, train_weight, bias, vals, keep_mask, rows, cols, users, pos, neg):
    rows = rows.astype(jnp.int32)
    cols = cols.astype(jnp.int32)
    users = users.astype(jnp.int32)
    pos = pos.astype(jnp.int32)
    neg = neg.astype(jnp.int32)

    gsrc = jnp.concatenate([cols, rows])
    sdst = jnp.concatenate([rows, cols])
    pack = jnp.bitwise_or(gsrc * DW, jnp.left_shift(sdst * DW, 16))
    tabi = node_emb[N_USERS:].reshape(N_ITEMS, NW, DW).transpose(1, 0, 2)
    tabu = node_emb[:N_USERS].reshape(N_USERS, NW, DW).transpose(1, 0, 2)
    tab = jnp.concatenate([tabi, tabu]).reshape(2 * NW * TABW)

    mesh = plsc.VectorSubcoreMesh(core_axis_name="c", subcore_axis_name="s")
    slabs = pl.kernel(
        _agg_body,
        out_type=jax.ShapeDtypeStruct((2 * NW * ACC,), jnp.float32),
        mesh=mesh,
        compiler_params=pltpu.CompilerParams(needs_layout_passes=False),
        scratch_types=[
            pltpu.VMEM((BLK,), jnp.int32),
            pltpu.VMEM((BLK,), jnp.float32),
            pltpu.VMEM((BLK,), jnp.float32),
            pltpu.VMEM((BLK,), jnp.float32),
            pltpu.VMEM((TABW,), jnp.float32),
            pltpu.VMEM((ACC,), jnp.float32),
        ],
    )(tab, pack, vals, keep_mask)

    red = slabs.reshape(2, NW, SEG_PAD, DW).transpose(0, 2, 1, 3).reshape(2, SEG_PAD, DIM)
    cat = jnp.concatenate([red[0, :N_USERS], red[1, :N_ITEMS]], axis=0)

    emb = pl.pallas_call(
        _emb_block,
        grid=(5,),
        in_specs=[
            pl.BlockSpec((N_NODES // 5, DIM), lambda i: (i, 0)),
            pl.BlockSpec((DIM, DIM), lambda i: (0, 0)),
            pl.BlockSpec((N_NODES // 5, DIM), lambda i: (i, 0)),
        ],
        out_specs=pl.BlockSpec((N_NODES // 5, DIM), lambda i: (i, 0)),
        out_shape=jax.ShapeDtypeStruct((N_NODES, DIM), jnp.float32),
    )(cat, train_weight, bias)

    pd, nd, sq = pl.kernel(
        _bpr_body,
        out_type=(
            jax.ShapeDtypeStruct((BATCH,), jnp.float32),
            jax.ShapeDtypeStruct((BATCH,), jnp.float32),
            jax.ShapeDtypeStruct((NC * NS * L,), jnp.float32),
        ),
        mesh=plsc.VectorSubcoreMesh(core_axis_name="c", subcore_axis_name="s"),
        compiler_params=pltpu.CompilerParams(needs_layout_passes=False),
        scratch_types=[
            pltpu.VMEM((BPB,), jnp.int32),
            pltpu.VMEM((BPB,), jnp.int32),
            pltpu.VMEM((BPB,), jnp.int32),
            pltpu.VMEM((BPB, DIM), jnp.float32),
            pltpu.VMEM((BPB, DIM), jnp.float32),
            pltpu.VMEM((BPB, DIM), jnp.float32),
            pltpu.VMEM((BPB,), jnp.float32),
            pltpu.VMEM((BPB,), jnp.float32),
            pltpu.VMEM((L,), jnp.float32),
            pltpu.SemaphoreType.DMA,
        ],
    )(emb, users, pos + N_USERS, neg + N_USERS)

    loss2 = pl.pallas_call(
        _loss_block,
        out_shape=jax.ShapeDtypeStruct((1, 1), jnp.float32),
    )(pd.reshape(32, 128), nd.reshape(32, 128), sq.reshape(4, 128))
    loss = loss2[0, 0]

    return (loss, emb[:N_USERS], emb[N_USERS:])


# speedup vs baseline: 1.3795x; 1.3795x over previous
import functools

import jax
import jax.numpy as jnp
from jax import lax
from jax.experimental import pallas as pl
from jax.experimental.pallas import tpu as pltpu
from jax.experimental.pallas import tpu_sc as plsc

N_USERS = 5000
N_ITEMS = 5000
N_NODES = N_USERS + N_ITEMS
DIM = 256
N_EDGES = 160000
DROP = 0.1
DECAY = 1e-4
BATCH = 4096

NC = 2
NS = 16
L = 16

NW = NC * NS
DW = DIM // NW
BLK = 8000
NBLK = N_EDGES // BLK
SEG_PAD = 5120
VG = DIM // L
ACC = SEG_PAD * DW
TABW = N_USERS * DW

BPB = BATCH // (NC * NS)


def _agg_body(tab_hbm, pack_hbm, vals_hbm, keep_hbm, out_hbm,
              psb, vsb, ksb, dvb, tabv, acc, sem):
    cid = lax.axis_index("c")
    sid = lax.axis_index("s")
    w = cid * NS + sid
    scale = 1.0 / (1.0 - DROP)

    for a in range(2):
        pltpu.sync_copy(tab_hbm.at[pl.ds((a * NW + w) * TABW, TABW)], tabv)

        def z_acc(i, carry):
            acc[pl.ds(i * L, L)] = jnp.zeros((L,), jnp.float32)
            return carry

        lax.fori_loop(0, ACC // L, z_acc, 0)

        def blk_body(b, carry):
            base = b * BLK
            c1 = pltpu.async_copy(pack_hbm.at[pl.ds(a * N_EDGES + base, BLK)], psb, sem)
            c2 = pltpu.async_copy(vals_hbm.at[pl.ds(base, BLK)], vsb, sem)
            c3 = pltpu.async_copy(keep_hbm.at[pl.ds(base, BLK)], ksb, sem)
            c1.wait()
            c2.wait()
            c3.wait()

            def dv_group(g, c2):
                s = pl.ds(g * L, L)
                dvb[s] = vsb[s] * ksb[s] * scale
                return c2

            lax.fori_loop(0, BLK // L, dv_group, 0)

            @plsc.parallel_loop(0, BLK // L, unroll=4)
            def grp_body(g):
                s = pl.ds(g * L, L)
                p16 = psb[s]
                dvv = dvb[s]
                sb8 = jnp.bitwise_and(p16, 0xFFFF)
                db8 = lax.shift_right_logical(p16, 16)
                for d in range(DW):
                    xv = plsc.load_gather(tabv, [sb8 + d])
                    plsc.addupdate_scatter(acc, [db8 + d], xv * dvv)
            return carry

        lax.fori_loop(0, NBLK, blk_body, 0)
        pltpu.sync_copy(acc, out_hbm.at[pl.ds((a * NW + w) * ACC, ACC)])


def _bpr_body(emb_hbm, uidx_hbm, pidx_hbm, nidx_hbm, pd_hbm, nd_hbm, sq_hbm,
              uidx_v, pidx_v, nidx_v, ubuf, pbuf, nbuf, pdv, ndv, sqv, sem):
    cid = lax.axis_index("c")
    sid = lax.axis_index("s")
    wid = sid * NC + cid
    base = wid * BPB

    pltpu.sync_copy(uidx_hbm.at[pl.ds(base, BPB)], uidx_v)
    pltpu.sync_copy(pidx_hbm.at[pl.ds(base, BPB)], pidx_v)
    pltpu.sync_copy(nidx_hbm.at[pl.ds(base, BPB)], nidx_v)
    c1 = pltpu.async_copy(emb_hbm.at[uidx_v], ubuf, sem)
    c2 = pltpu.async_copy(emb_hbm.at[pidx_v], pbuf, sem)
    c3 = pltpu.async_copy(emb_hbm.at[nidx_v], nbuf, sem)
    c1.wait()
    c2.wait()
    c3.wait()

    lane = lax.iota(jnp.int32, L)
    zero = jnp.zeros((L,), jnp.float32)

    def body(i, carry):
        sacc, pvec, nvec = carry
        pacc = zero
        nacc = zero
        for g in range(VG):
            s = pl.ds(g * L, L)
            u = ubuf[i, s]
            p = pbuf[i, s]
            n = nbuf[i, s]
            pacc = pacc + u * p
            nacc = nacc + u * n
            sacc = sacc + u * u + p * p + n * n
        ps = jnp.sum(pacc)
        ns = jnp.sum(nacc)
        j = jnp.bitwise_and(i, L - 1)
        pvec = jnp.where(lane == j, ps, pvec)
        nvec = jnp.where(lane == j, ns, nvec)

        @pl.when(j == L - 1)
        def _flush():
            st = i - (L - 1)
            pdv[pl.ds(st, L)] = pvec
            ndv[pl.ds(st, L)] = nvec

        return (sacc, pvec, nvec)

    sacc, _, _ = lax.fori_loop(0, BPB, body, (zero, zero, zero))
    sqv[...] = sacc
    pltpu.sync_copy(pdv, pd_hbm.at[pl.ds(base, BPB)])
    pltpu.sync_copy(ndv, nd_hbm.at[pl.ds(base, BPB)])
    pltpu.sync_copy(sqv, sq_hbm.at[pl.ds(wid * L, L)])


def _slab_reduce_block(x_ref, o_ref):
    o_ref[...] = jnp.sum(x_ref[0], axis=0)[None]


def _emb_block(x_ref, w_ref, b_ref, o_ref):
    x = x_ref[...]
    y = jnp.dot(x, w_ref[...], preferred_element_type=jnp.float32) + b_ref[...]
    y = jnp.where(y >= 0, y, 0.2 * y)
    n = jnp.sqrt(jnp.sum(y * y, axis=1, keepdims=True))
    o_ref[...] = y / jnp.maximum(n, 1e-12)


def _loss_block(pd_ref, nd_ref, sq_ref, o_ref):
    d = pd_ref[...] - nd_ref[...]
    z = -d
    sp = jnp.maximum(z, 0.0) + jnp.log1p(jnp.exp(-jnp.abs(z)))
    mf = jnp.sum(sp) * (1.0 / BATCH)
    reg = jnp.sum(sq_ref[...]) * 0.5
    o_ref[...] = jnp.broadcast_to(mf + DECAY * reg * (1.0 / BATCH), (1, 1))


def kernel(node_emb, train_weight, bias, vals, keep_mask, rows, cols, users, pos, neg):
    rows = rows.astype(jnp.int32)
    cols = cols.astype(jnp.int32)
    users = users.astype(jnp.int32)
    pos = pos.astype(jnp.int32)
    neg = neg.astype(jnp.int32)

    gsrc = jnp.concatenate([cols, rows])
    sdst = jnp.concatenate([rows, cols])
    pack = jnp.bitwise_or(gsrc * DW, jnp.left_shift(sdst * DW, 16))
    tabi = node_emb[N_USERS:].reshape(N_ITEMS, NW, DW).transpose(1, 0, 2)
    tabu = node_emb[:N_USERS].reshape(N_USERS, NW, DW).transpose(1, 0, 2)
    tab = jnp.concatenate([tabi, tabu]).reshape(2 * NW * TABW)

    mesh = plsc.VectorSubcoreMesh(core_axis_name="c", subcore_axis_name="s")
    slabs = pl.kernel(
        _agg_body,
        out_type=jax.ShapeDtypeStruct((2 * NW * ACC,), jnp.float32),
        mesh=mesh,
        compiler_params=pltpu.CompilerParams(needs_layout_passes=False),
        scratch_types=[
            pltpu.VMEM((BLK,), jnp.int32),
            pltpu.VMEM((BLK,), jnp.float32),
            pltpu.VMEM((BLK,), jnp.float32),
            pltpu.VMEM((BLK,), jnp.float32),
            pltpu.VMEM((TABW,), jnp.float32),
            pltpu.VMEM((ACC,), jnp.float32),
            pltpu.SemaphoreType.DMA,
        ],
    )(tab, pack, vals, keep_mask)

    red = slabs.reshape(2, NW, SEG_PAD, DW).transpose(0, 2, 1, 3).reshape(2, SEG_PAD, DIM)
    cat = jnp.concatenate([red[0, :N_USERS], red[1, :N_ITEMS]], axis=0)

    emb = pl.pallas_call(
        _emb_block,
        grid=(5,),
        in_specs=[
            pl.BlockSpec((N_NODES // 5, DIM), lambda i: (i, 0)),
            pl.BlockSpec((DIM, DIM), lambda i: (0, 0)),
            pl.BlockSpec((N_NODES // 5, DIM), lambda i: (i, 0)),
        ],
        out_specs=pl.BlockSpec((N_NODES // 5, DIM), lambda i: (i, 0)),
        out_shape=jax.ShapeDtypeStruct((N_NODES, DIM), jnp.float32),
    )(cat, train_weight, bias)

    pd, nd, sq = pl.kernel(
        _bpr_body,
        out_type=(
            jax.ShapeDtypeStruct((BATCH,), jnp.float32),
            jax.ShapeDtypeStruct((BATCH,), jnp.float32),
            jax.ShapeDtypeStruct((NC * NS * L,), jnp.float32),
        ),
        mesh=plsc.VectorSubcoreMesh(core_axis_name="c", subcore_axis_name="s"),
        compiler_params=pltpu.CompilerParams(needs_layout_passes=False),
        scratch_types=[
            pltpu.VMEM((BPB,), jnp.int32),
            pltpu.VMEM((BPB,), jnp.int32),
            pltpu.VMEM((BPB,), jnp.int32),
            pltpu.VMEM((BPB, DIM), jnp.float32),
            pltpu.VMEM((BPB, DIM), jnp.float32),
            pltpu.VMEM((BPB, DIM), jnp.float32),
            pltpu.VMEM((BPB,), jnp.float32),
            pltpu.VMEM((BPB,), jnp.float32),
            pltpu.VMEM((L,), jnp.float32),
            pltpu.SemaphoreType.DMA,
        ],
    )(emb, users, pos + N_USERS, neg + N_USERS)

    loss2 = pl.pallas_call(
        _loss_block,
        out_shape=jax.ShapeDtypeStruct((1, 1), jnp.float32),
    )(pd.reshape(32, 128), nd.reshape(32, 128), sq.reshape(4, 128))
    loss = loss2[0, 0]

    return (loss, emb[:N_USERS], emb[N_USERS:])
